# baseline (speedup 1.0000x reference)
import functools

import jax
import jax.numpy as jnp
from jax import lax
from jax.experimental import pallas as pl
from jax.experimental.pallas import tpu as pltpu
from jax.experimental.pallas import tpu_sc as plsc

D_MODEL = 64
SCALE = 8.0
NUM_WORKERS = 32
LANES = 16


def kernel(input, table):
    batch, hist = input.shape
    n_ht = hist // 8
    n_bt = batch // 128
    p_in = input.T.reshape(n_ht, 8, 128, 128).transpose(0, 2, 1, 3)
    if p_in.dtype != jnp.int32:
        p_in = p_in.astype(jnp.int32)

    n_units = n_ht * n_bt * 4
    units_per_w = n_units // NUM_WORKERS
    assert units_per_w * NUM_WORKERS == n_units and units_per_w % 2 == 0
    n2 = units_per_w // 2

    mesh = plsc.VectorSubcoreMesh(core_axis_name="c", subcore_axis_name="s")

    @functools.partial(
        pl.kernel,
        mesh=mesh,
        out_type=jax.ShapeDtypeStruct((hist, 8, n_bt, 8, 128), jnp.float32),
        scratch_types=[
            pltpu.VMEM((2, 2, 128), jnp.int32),
            pltpu.VMEM((2, 2, 128, D_MODEL), jnp.float32),
            pltpu.VMEM((2, 2, 8, 8, 128), jnp.float32),
            pltpu.SemaphoreType.DMA,
            pltpu.SemaphoreType.DMA,
            pltpu.SemaphoreType.DMA,
            pltpu.SemaphoreType.DMA,
            pltpu.SemaphoreType.DMA,
            pltpu.SemaphoreType.DMA,
        ],
        compiler_params=pltpu.CompilerParams(use_tc_tiling_on_sc=False,
                                             needs_layout_passes=False),
    )
    def emb(idx_hbm, table_hbm, out_hbm, idx_v, rows_v, trans_v,
            isem0, isem1, gsem0, gsem1, ssem0, ssem1):
        isems = [isem0, isem1]
        gsems = [gsem0, gsem1]
        ssems = [ssem0, ssem1]
        wid = lax.axis_index("s") * 2 + lax.axis_index("c")
        u0 = wid * units_per_w

        def unit_coords(u):
            ht = u >> 9
            bt = (u >> 2) & 127
            hs0 = (u & 3) * 2
            return ht, bt, hs0

        def fire_idx(u, s):
            ht, bt, hs0 = unit_coords(u)
            pltpu.async_copy(idx_hbm.at[ht, bt, pl.ds(hs0, 2)],
                             idx_v.at[s], isems[s])

        def wait_idx(s):
            pltpu.make_async_copy(idx_hbm.at[0, 0, pl.ds(0, 2)],
                                  idx_v.at[s], isems[s]).wait()

        def fire_gathers(s):
            for hl in range(2):
                pltpu.async_copy(table_hbm.at[idx_v.at[s, hl]],
                                 rows_v.at[s, hl], gsems[s])

        def wait_gathers(s):
            for hl in range(2):
                pltpu.make_async_copy(table_hbm.at[idx_v.at[s, hl]],
                                      rows_v.at[s, hl], gsems[s]).wait()

        def transpose_scale(s):
            iota = lax.iota(jnp.int32, LANES)
            c_s = jnp.full((LANES,), s, jnp.int32)

            def body(blc, c):
                blv = blc * LANES + iota
                for hl in range(2):
                    c_hl = jnp.full((LANES,), hl, jnp.int32)
                    for d in range(D_MODEL):
                        vals = plsc.load_gather(
                            rows_v,
                            [c_s, c_hl, blv, jnp.full((LANES,), d, jnp.int32)])
                        trans_v[s, hl, d // 8, d % 8,
                                pl.ds(blc * LANES, LANES)] = vals * SCALE
                return c

            lax.fori_loop(0, 128 // LANES, body, 0)

        def fire_stores(u, s):
            ht, bt, hs0 = unit_coords(u)
            for hl in range(2):
                h = ht * 8 + hs0 + hl
                for dt in range(8):
                    pltpu.async_copy(trans_v.at[s, hl, dt],
                                     out_hbm.at[h, dt, bt], ssems[s])

        def wait_stores(s):
            for _ in range(16):
                pltpu.make_async_copy(trans_v.at[s, 0, 0],
                                      out_hbm.at[0, 0, 0], ssems[s]).wait()

        fire_idx(u0, 0)
        fire_idx(u0 + 1, 1)
        wait_idx(0)
        fire_gathers(0)

        def outer(p, c):
            u = u0 + 2 * p
            wait_gathers(0)

            @pl.when(p < n2 - 1)
            def _():
                fire_idx(u + 2, 0)

            wait_idx(1)
            fire_gathers(1)

            @pl.when(p >= 1)
            def _():
                wait_stores(0)

            transpose_scale(0)
            fire_stores(u, 0)

            wait_gathers(1)

            @pl.when(p < n2 - 1)
            def _():
                fire_idx(u + 3, 1)
                wait_idx(0)
                fire_gathers(0)

            @pl.when(p >= 1)
            def _():
                wait_stores(1)

            transpose_scale(1)
            fire_stores(u + 1, 1)
            return c

        lax.fori_loop(0, n2, outer, 0)
        wait_stores(0)
        wait_stores(1)

    x = emb(p_in, table)
    return x.transpose(2, 4, 0, 1, 3).reshape(batch, hist, D_MODEL)


# speedup vs baseline: 1.5598x; 1.5598x over previous
import functools

import jax
import jax.numpy as jnp
from jax import lax
from jax.experimental import pallas as pl
from jax.experimental.pallas import tpu as pltpu
from jax.experimental.pallas import tpu_sc as plsc

D_MODEL = 64
SCALE = 8.0
NUM_WORKERS = 32
LANES = 16


def kernel(input, table):
    batch, hist = input.shape
    n_ht = hist // 8
    n_bt = batch // 128
    p_in = input.T.reshape(n_ht, 8, 128, 128).transpose(0, 2, 1, 3)
    if p_in.dtype != jnp.int32:
        p_in = p_in.astype(jnp.int32)

    n_units = n_ht * n_bt * 4
    units_per_w = n_units // NUM_WORKERS
    assert units_per_w * NUM_WORKERS == n_units and units_per_w % 2 == 0
    n2 = units_per_w // 2

    mesh = plsc.VectorSubcoreMesh(core_axis_name="c", subcore_axis_name="s")

    @functools.partial(
        pl.kernel,
        mesh=mesh,
        out_type=jax.ShapeDtypeStruct((hist, 8, n_bt, 8, 128), jnp.float32),
        scratch_types=[
            pltpu.VMEM((2, 2, 128), jnp.int32),
            pltpu.VMEM((2, 2, 128, D_MODEL), jnp.float32),
            pltpu.VMEM((2, 2, 8, 8, 128), jnp.float32),
            pltpu.SemaphoreType.DMA,
            pltpu.SemaphoreType.DMA,
            pltpu.SemaphoreType.DMA,
            pltpu.SemaphoreType.DMA,
            pltpu.SemaphoreType.DMA,
            pltpu.SemaphoreType.DMA,
        ],
        compiler_params=pltpu.CompilerParams(use_tc_tiling_on_sc=False,
                                             needs_layout_passes=False),
    )
    def emb(idx_hbm, table_hbm, out_hbm, idx_v, rows_v, trans_v,
            isem0, isem1, gsem0, gsem1, ssem0, ssem1):
        isems = [isem0, isem1]
        gsems = [gsem0, gsem1]
        ssems = [ssem0, ssem1]
        wid = lax.axis_index("s") * 2 + lax.axis_index("c")
        u0 = wid * units_per_w

        def unit_coords(u):
            ht = u >> 9
            bt = (u >> 2) & 127
            hs0 = (u & 3) * 2
            return ht, bt, hs0

        def fire_idx(u, s):
            ht, bt, hs0 = unit_coords(u)
            pltpu.async_copy(idx_hbm.at[ht, bt, pl.ds(hs0, 2)],
                             idx_v.at[s], isems[s])

        def wait_idx(s):
            pltpu.make_async_copy(idx_hbm.at[0, 0, pl.ds(0, 2)],
                                  idx_v.at[s], isems[s]).wait()

        def fire_gathers(s):
            for hl in range(2):
                pltpu.async_copy(table_hbm.at[idx_v.at[s, hl]],
                                 rows_v.at[s, hl], gsems[s])

        def wait_gathers(s):
            for hl in range(2):
                pltpu.make_async_copy(table_hbm.at[idx_v.at[s, hl]],
                                      rows_v.at[s, hl], gsems[s]).wait()

        def transpose_scale(s):
            iota = lax.iota(jnp.int32, LANES)
            c_s = jnp.full((LANES,), s, jnp.int32)

            @plsc.parallel_loop(0, 128 // LANES)
            def body(blc):
                blv = blc * LANES + iota
                for hl in range(2):
                    c_hl = jnp.full((LANES,), hl, jnp.int32)
                    for d in range(D_MODEL):
                        vals = plsc.load_gather(
                            rows_v,
                            [c_s, c_hl, blv, jnp.full((LANES,), d, jnp.int32)])
                        trans_v[s, hl, d // 8, d % 8,
                                pl.ds(blc * LANES, LANES)] = vals * SCALE

        def fire_stores(u, s):
            ht, bt, hs0 = unit_coords(u)
            for hl in range(2):
                h = ht * 8 + hs0 + hl
                for dt in range(8):
                    pltpu.async_copy(trans_v.at[s, hl, dt],
                                     out_hbm.at[h, dt, bt], ssems[s])

        def wait_stores(s):
            for _ in range(16):
                pltpu.make_async_copy(trans_v.at[s, 0, 0],
                                      out_hbm.at[0, 0, 0], ssems[s]).wait()

        fire_idx(u0, 0)
        fire_idx(u0 + 1, 1)
        wait_idx(0)
        fire_gathers(0)

        def outer(p, c):
            u = u0 + 2 * p
            wait_gathers(0)

            @pl.when(p < n2 - 1)
            def _():
                fire_idx(u + 2, 0)

            wait_idx(1)
            fire_gathers(1)

            @pl.when(p >= 1)
            def _():
                wait_stores(0)

            transpose_scale(0)
            fire_stores(u, 0)

            wait_gathers(1)

            @pl.when(p < n2 - 1)
            def _():
                fire_idx(u + 3, 1)
                wait_idx(0)
                fire_gathers(0)

            @pl.when(p >= 1)
            def _():
                wait_stores(1)

            transpose_scale(1)
            fire_stores(u + 1, 1)
            return c

        lax.fori_loop(0, n2, outer, 0)
        wait_stores(0)
        wait_stores(1)

    x = emb(p_in, table)
    return x.transpose(2, 4, 0, 1, 3).reshape(batch, hist, D_MODEL)


# speedup vs baseline: 4.7780x; 3.0633x over previous
import functools

import jax
import jax.numpy as jnp
from jax import lax
from jax.experimental import pallas as pl
from jax.experimental.pallas import tpu as pltpu
from jax.experimental.pallas import tpu_sc as plsc

D_MODEL = 64
SCALE = 8.0
NUM_WORKERS = 32
LANES = 16
TPAD = 131


def kernel(input, table):
    batch, hist = input.shape
    n_ht = hist // 8
    n_bt = batch // 128
    p_in = input.T.reshape(n_ht, 8, 128, 128).transpose(0, 2, 1, 3)
    if p_in.dtype != jnp.int32:
        p_in = p_in.astype(jnp.int32)

    n_units = n_ht * n_bt * 4
    units_per_w = n_units // NUM_WORKERS
    assert units_per_w * NUM_WORKERS == n_units and units_per_w % 2 == 0
    n2 = units_per_w // 2

    mesh = plsc.VectorSubcoreMesh(core_axis_name="c", subcore_axis_name="s")

    @functools.partial(
        pl.kernel,
        mesh=mesh,
        out_type=jax.ShapeDtypeStruct((hist, 8, n_bt, 8, 128), jnp.float32),
        scratch_types=[
            pltpu.VMEM((2, 2, 128), jnp.int32),
            pltpu.VMEM((2 * 2 * 128, D_MODEL), jnp.float32),
            pltpu.VMEM((2 * 2 * D_MODEL, TPAD), jnp.float32),
            pltpu.SemaphoreType.DMA,
            pltpu.SemaphoreType.DMA,
            pltpu.SemaphoreType.DMA,
            pltpu.SemaphoreType.DMA,
            pltpu.SemaphoreType.DMA,
            pltpu.SemaphoreType.DMA,
        ],
        compiler_params=pltpu.CompilerParams(use_tc_tiling_on_sc=False,
                                             needs_layout_passes=False),
    )
    def emb(idx_hbm, table_hbm, out_hbm, idx_v, rows_v, trans_v,
            isem0, isem1, gsem0, gsem1, ssem0, ssem1):
        isems = [isem0, isem1]
        gsems = [gsem0, gsem1]
        ssems = [ssem0, ssem1]
        wid = lax.axis_index("s") * 2 + lax.axis_index("c")
        u0 = wid * units_per_w

        def unit_coords(u):
            ht = u >> 9
            bt = (u >> 2) & 127
            hs0 = (u & 3) * 2
            return ht, bt, hs0

        def fire_idx(u, s):
            ht, bt, hs0 = unit_coords(u)
            pltpu.async_copy(idx_hbm.at[ht, bt, pl.ds(hs0, 2)],
                             idx_v.at[s], isems[s])

        def wait_idx(s):
            pltpu.make_async_copy(idx_hbm.at[0, 0, pl.ds(0, 2)],
                                  idx_v.at[s], isems[s]).wait()

        def fire_gathers(s):
            for hl in range(2):
                pltpu.async_copy(table_hbm.at[idx_v.at[s, hl]],
                                 rows_v.at[pl.ds((s * 2 + hl) * 128, 128)],
                                 gsems[s])

        def wait_gathers(s):
            for hl in range(2):
                pltpu.make_async_copy(
                    table_hbm.at[idx_v.at[s, hl]],
                    rows_v.at[pl.ds((s * 2 + hl) * 128, 128)],
                    gsems[s]).wait()

        def transpose_scale(s):
            iota = lax.iota(jnp.int32, LANES)

            @plsc.parallel_loop(0, 128)
            def body(bl):
                c_bl = jnp.full((LANES,), bl, jnp.int32)
                for hl in range(2):
                    rbase = (s * 2 + hl) * 128
                    tbase = (s * 2 + hl) * D_MODEL
                    for dg in range(D_MODEL // LANES):
                        vals = rows_v[rbase + bl, pl.ds(dg * LANES, LANES)]
                        rowv = (tbase + dg * LANES) + iota
                        plsc.store_scatter(trans_v, [rowv, c_bl],
                                           vals * SCALE)

        def fire_stores(u, s):
            ht, bt, hs0 = unit_coords(u)
            for hl in range(2):
                h = ht * 8 + hs0 + hl
                tbase = (s * 2 + hl) * D_MODEL
                for dt in range(8):
                    pltpu.async_copy(
                        trans_v.at[pl.ds(tbase + dt * 8, 8), pl.ds(0, 128)],
                        out_hbm.at[h, dt, bt], ssems[s])

        def wait_stores(s):
            for _ in range(16):
                pltpu.make_async_copy(
                    trans_v.at[pl.ds(0, 8), pl.ds(0, 128)],
                    out_hbm.at[0, 0, 0], ssems[s]).wait()

        fire_idx(u0, 0)
        fire_idx(u0 + 1, 1)
        wait_idx(0)
        fire_gathers(0)

        def outer(p, c):
            u = u0 + 2 * p
            wait_gathers(0)

            @pl.when(p < n2 - 1)
            def _():
                fire_idx(u + 2, 0)

            wait_idx(1)
            fire_gathers(1)

            @pl.when(p >= 1)
            def _():
                wait_stores(0)

            transpose_scale(0)
            fire_stores(u, 0)

            wait_gathers(1)

            @pl.when(p < n2 - 1)
            def _():
                fire_idx(u + 3, 1)
                wait_idx(0)
                fire_gathers(0)

            @pl.when(p >= 1)
            def _():
                wait_stores(1)

            transpose_scale(1)
            fire_stores(u + 1, 1)
            return c

        lax.fori_loop(0, n2, outer, 0)
        wait_stores(0)
        wait_stores(1)

    x = emb(p_in, table)
    return x.transpose(2, 4, 0, 1, 3).reshape(batch, hist, D_MODEL)
